# baseline (speedup 1.0000x reference)
import functools

import jax
import jax.numpy as jnp
from jax.experimental import pallas as pl
from jax.experimental.pallas import tpu as pltpu

N = 16384
C = 1000
RATE = 0.8
K = min(N, int(N * RATE))
BR = 1024
NB = N // BR


def _ohem_body(x_ref, t_ref, out_ref, loss_scr):
    i = pl.program_id(0)
    x = x_ref[...]
    t = t_ref[0, 0, :]
    col = jax.lax.broadcasted_iota(jnp.int32, (BR, C), 1)
    onehot = col == t[:, None]
    e = jnp.exp(x)
    s = jnp.sum(e, axis=1)
    tval = jnp.sum(jnp.where(onehot, x, 0.0), axis=1)
    loss = jnp.maximum(jnp.log(s) - tval, 0.0)
    loss = jnp.where(t == -1, 0.0, loss)
    loss_scr[i, :] = loss

    @pl.when(i == NB - 1)
    def _select():
        v = loss_scr[...]
        u = jax.lax.bitcast_convert_type(v, jnp.int32)

        def body(_, lo_hi):
            lo, hi = lo_hi
            mid = lo + ((hi - lo + 1) >> 1)
            cnt = jnp.sum((u >= mid).astype(jnp.int32))
            ge = cnt >= K
            return jnp.where(ge, mid, lo), jnp.where(ge, hi, mid - 1)

        lo, _ = jax.lax.fori_loop(
            0, 31, body, (jnp.int32(0), jnp.int32(0x7F7FFFFF)))
        t_kth = jax.lax.bitcast_convert_type(lo, jnp.float32)
        gt = u > lo
        c_gt = jnp.sum(gt.astype(jnp.int32))
        s_gt = jnp.sum(jnp.where(gt, v, 0.0))
        out_ref[0, 0] = (s_gt + (K - c_gt).astype(jnp.float32) * t_kth) / K


@jax.jit
def _ohem(cls_pred, tgt3):
    out = pl.pallas_call(
        _ohem_body,
        grid=(NB,),
        in_specs=[
            pl.BlockSpec((BR, C), lambda i: (i, 0)),
            pl.BlockSpec((1, 1, BR), lambda i: (i, 0, 0)),
        ],
        out_specs=pl.BlockSpec(
            (1, 1), lambda i: (0, 0), memory_space=pltpu.SMEM),
        out_shape=jax.ShapeDtypeStruct((1, 1), jnp.float32),
        scratch_shapes=[pltpu.VMEM((NB, BR), jnp.float32)],
    )(cls_pred, tgt3)
    return out[0, 0]


def kernel(cls_pred, cls_target):
    tgt3 = cls_target.astype(jnp.int32).reshape(NB, 1, BR)
    return _ohem(cls_pred, tgt3)


# speedup vs baseline: 1.0351x; 1.0351x over previous
import jax
import jax.numpy as jnp
from jax.experimental import pallas as pl
from jax.experimental.pallas import tpu as pltpu

N = 16384
C = 1000
RATE = 0.8
K = min(N, int(N * RATE))
BR = 1024
NB = N // BR
NB2 = NB // 2


def _ce_rows(x, t):
    col = jax.lax.broadcasted_iota(jnp.int32, (BR, C), 1)
    onehot = col == t[:, None]
    s = jnp.sum(jnp.exp(x), axis=1)
    tval = jnp.sum(jnp.where(onehot, x, 0.0), axis=1)
    loss = jnp.maximum(jnp.log(s) - tval, 0.0)
    return jnp.where(t == -1, 0.0, loss)


def _ohem_body(x0_ref, x1_ref, t0_ref, t1_ref, out_ref, loss_scr):
    i = pl.program_id(0)
    loss_scr[i, :] = _ce_rows(x0_ref[...], t0_ref[0, 0, :])
    loss_scr[i + NB2, :] = _ce_rows(x1_ref[...], t1_ref[0, 0, :])

    @pl.when(i == NB2 - 1)
    def _select():
        v = loss_scr[...]
        u = jax.lax.bitcast_convert_type(v, jnp.int32)

        def body(_, lo_hi):
            lo, hi = lo_hi
            mid = lo + ((hi - lo + 1) >> 1)
            cnt = jnp.sum((u >= mid).astype(jnp.int32))
            ge = cnt >= K
            return jnp.where(ge, mid, lo), jnp.where(ge, hi, mid - 1)

        lo, _ = jax.lax.fori_loop(
            0, 31, body, (jnp.int32(0), jnp.int32(0x7F7FFFFF)))
        t_kth = jax.lax.bitcast_convert_type(lo, jnp.float32)
        gt = u > lo
        c_gt = jnp.sum(gt.astype(jnp.int32))
        s_gt = jnp.sum(jnp.where(gt, v, 0.0))
        out_ref[0, 0] = (s_gt + (K - c_gt).astype(jnp.float32) * t_kth) / K


@jax.jit
def _ohem(cls_pred, tgt3):
    out = pl.pallas_call(
        _ohem_body,
        grid=(NB2,),
        in_specs=[
            pl.BlockSpec((BR, C), lambda i: (i, 0)),
            pl.BlockSpec((BR, C), lambda i: (i + NB2, 0)),
            pl.BlockSpec((1, 1, BR), lambda i: (i, 0, 0)),
            pl.BlockSpec((1, 1, BR), lambda i: (i + NB2, 0, 0)),
        ],
        out_specs=pl.BlockSpec(
            (1, 1), lambda i: (0, 0), memory_space=pltpu.SMEM),
        out_shape=jax.ShapeDtypeStruct((1, 1), jnp.float32),
        scratch_shapes=[pltpu.VMEM((NB, BR), jnp.float32)],
    )(cls_pred, cls_pred, tgt3, tgt3)
    return out[0, 0]


def kernel(cls_pred, cls_target):
    tgt3 = cls_target.astype(jnp.int32).reshape(NB, 1, BR)
    return _ohem(cls_pred, tgt3)
